# baseline (speedup 1.0000x reference)
import functools

import jax
import jax.numpy as jnp
from jax import lax
from jax.experimental import pallas as pl
from jax.experimental.pallas import tpu as pltpu
from jax.experimental.pallas import tpu_sc as plsc

B = 16384
F = 128
H = 32
E = 16
N_ROWS = 1000000
G = 128 // E
N_GRP = N_ROWS // G
NC, NS, L = 2, 16, 16
NW = NC * NS
BPW = B // NW
CH = 128
NCHUNK = BPW // CH
CROWS = BPW * E // 128
MLP_BB = 2048


def _mlp_body(uf_ref, w1_ref, b1_ref, w2_ref, b2_ref, out_ref):
    h = jnp.dot(uf_ref[...], w1_ref[...], preferred_element_type=jnp.float32)
    h = jnp.maximum(h + b1_ref[...], 0.0)
    out_ref[...] = (
        jnp.dot(h, w2_ref[...], preferred_element_type=jnp.float32) + b2_ref[...]
    )


_mlp = pl.pallas_call(
    _mlp_body,
    grid=(B // MLP_BB,),
    in_specs=[
        pl.BlockSpec((MLP_BB, F), lambda i: (i, 0)),
        pl.BlockSpec((F, H), lambda i: (0, 0)),
        pl.BlockSpec((1, H), lambda i: (0, 0)),
        pl.BlockSpec((H, E), lambda i: (0, 0)),
        pl.BlockSpec((1, E), lambda i: (0, 0)),
    ],
    out_specs=pl.BlockSpec((MLP_BB, E), lambda i: (i, 0)),
    out_shape=jax.ShapeDtypeStruct((B, E), jnp.float32),
)


@functools.cache
def _make_sc_score():
    mesh = plsc.VectorSubcoreMesh(
        core_axis_name="c", subcore_axis_name="s", num_cores=NC, num_subcores=NS
    )

    @functools.partial(
        pl.kernel,
        out_type=jax.ShapeDtypeStruct((B,), jnp.float32),
        mesh=mesh,
        compiler_params=pltpu.CompilerParams(needs_layout_passes=False),
        scratch_types=[
            pltpu.VMEM((BPW,), jnp.int32),
            pltpu.VMEM((BPW,), jnp.int32),
            pltpu.VMEM((BPW,), jnp.int32),
            pltpu.VMEM((BPW,), jnp.int32),
            pltpu.VMEM((CH, 128), jnp.float32),
            pltpu.VMEM((CH, 128), jnp.float32),
            pltpu.VMEM((CH, 128), jnp.float32),
            pltpu.VMEM((CH, 128), jnp.float32),
            pltpu.VMEM((CROWS, 128), jnp.float32),
            pltpu.VMEM((BPW,), jnp.float32),
            pltpu.SemaphoreType.DMA,
            pltpu.SemaphoreType.DMA,
            pltpu.SemaphoreType.DMA,
            pltpu.SemaphoreType.DMA,
        ],
    )
    def _sc_score(uid_hbm, iid_hbm, content_hbm, utab_hbm, itab_hbm, out_hbm,
                  uidx_v, iidx_v, ugrp_v, igrp_v,
                  ub0, ub1, ib0, ib1, c_v, s_v,
                  us0, us1, is0, is1):
        wid = lax.axis_index("s") * NC + lax.axis_index("c")
        base = wid * BPW
        pltpu.sync_copy(uid_hbm.at[pl.ds(base, BPW)], uidx_v)
        pltpu.sync_copy(iid_hbm.at[pl.ds(base, BPW)], iidx_v)
        pltpu.sync_copy(content_hbm.at[pl.ds(wid * CROWS, CROWS), :], c_v)

        def grp_body(k, carry):
            sl = pl.ds(k * L, L)
            ugrp_v[sl] = jnp.right_shift(uidx_v[sl], 3)
            igrp_v[sl] = jnp.right_shift(iidx_v[sl], 3)
            return carry

        lax.fori_loop(0, BPW // L, grp_body, 0)

        ubufs, ibufs = [ub0, ub1], [ib0, ib1]
        usems, isems = [us0, us1], [is0, is1]

        def fire(k):
            slot = k % 2
            cu = pltpu.async_copy(
                utab_hbm.at[ugrp_v.at[pl.ds(k * CH, CH)]], ubufs[slot], usems[slot])
            ci = pltpu.async_copy(
                itab_hbm.at[igrp_v.at[pl.ds(k * CH, CH)]], ibufs[slot], isems[slot])
            return cu, ci

        lanes = lax.iota(jnp.int32, L)
        pending = fire(0)
        for k in range(NCHUNK):
            nxt = fire(k + 1) if k + 1 < NCHUNK else None
            pending[0].wait()
            pending[1].wait()
            bu, bi = ubufs[k % 2], ibufs[k % 2]

            def blk_body(blk, carry, k=k, bu=bu, bi=bi):
                bls = blk * L + lanes
                bs = k * CH + bls
                idu = uidx_v[pl.ds(k * CH + blk * L, L)]
                idi = iidx_v[pl.ds(k * CH + blk * L, L)]
                offu = (idu & (G - 1)) * E
                offi = (idi & (G - 1)) * E
                crow = jnp.right_shift(bs, 3)
                ccol = (bs & 7) * E
                acc = jnp.zeros((L,), jnp.float32)
                for e in range(E):
                    u = plsc.load_gather(bu, [bls, offu + e])
                    it = plsc.load_gather(bi, [bls, offi + e])
                    c = plsc.load_gather(c_v, [crow, ccol + e])
                    acc = acc + (u + c) * it
                s_v[pl.ds(k * CH + blk * L, L)] = acc
                return carry

            lax.fori_loop(0, CH // L, blk_body, 0)
            pending = nxt

        pltpu.sync_copy(s_v, out_hbm.at[pl.ds(base, BPW)])

    return _sc_score


def kernel(user_ids, item_ids, user_features, cf_user_table, cf_item_table,
           W1, b1, W2, b2):
    content = _mlp(user_features, W1, b1.reshape(1, H), W2, b2.reshape(1, E))
    return _make_sc_score()(user_ids, item_ids,
                            content.reshape(B * E // 128, 128),
                            cf_user_table.reshape(N_GRP, 128),
                            cf_item_table.reshape(N_GRP, 128))


# speedup vs baseline: 1.0115x; 1.0115x over previous
import functools

import jax
import jax.numpy as jnp
from jax import lax
from jax.experimental import pallas as pl
from jax.experimental.pallas import tpu as pltpu
from jax.experimental.pallas import tpu_sc as plsc

B = 16384
F = 128
H = 32
E = 16
N_ROWS = 1000000
NC, NS, L = 2, 16, 16
NW = NC * NS
BPW = B // NW
MLP_BB = 2048


def _mlp_body(uf_ref, w1_ref, b1_ref, w2_ref, b2_ref, out_ref):
    h = jnp.dot(uf_ref[...], w1_ref[...], preferred_element_type=jnp.float32)
    h = jnp.maximum(h + b1_ref[...], 0.0)
    out_ref[...] = (
        jnp.dot(h, w2_ref[...], preferred_element_type=jnp.float32) + b2_ref[...]
    )


_mlp = pl.pallas_call(
    _mlp_body,
    grid=(B // MLP_BB,),
    in_specs=[
        pl.BlockSpec((MLP_BB, F), lambda i: (i, 0)),
        pl.BlockSpec((F, H), lambda i: (0, 0)),
        pl.BlockSpec((1, H), lambda i: (0, 0)),
        pl.BlockSpec((H, E), lambda i: (0, 0)),
        pl.BlockSpec((1, E), lambda i: (0, 0)),
    ],
    out_specs=pl.BlockSpec((MLP_BB, E), lambda i: (i, 0)),
    out_shape=jax.ShapeDtypeStruct((B, E), jnp.float32),
)


@functools.cache
def _make_sc_score():
    mesh = plsc.VectorSubcoreMesh(
        core_axis_name="c", subcore_axis_name="s", num_cores=NC, num_subcores=NS
    )

    @functools.partial(
        pl.kernel,
        out_type=jax.ShapeDtypeStruct((B,), jnp.float32),
        mesh=mesh,
        compiler_params=pltpu.CompilerParams(
            needs_layout_passes=False,
            use_tc_tiling_on_sc=False,
            skip_device_barrier=True,
        ),
        scratch_types=[
            pltpu.VMEM((BPW,), jnp.int32),
            pltpu.VMEM((BPW,), jnp.int32),
            pltpu.VMEM((BPW, E), jnp.float32),
            pltpu.VMEM((BPW, E), jnp.float32),
            pltpu.VMEM((BPW, E), jnp.float32),
            pltpu.VMEM((BPW,), jnp.float32),
            pltpu.SemaphoreType.DMA,
            pltpu.SemaphoreType.DMA,
        ],
    )
    def _sc_score(uid_hbm, iid_hbm, content_hbm, utab_hbm, itab_hbm, out_hbm,
                  uidx_v, iidx_v, urows_v, irows_v, c_v, s_v, usem, isem):
        wid = lax.axis_index("s") * NC + lax.axis_index("c")
        base = wid * BPW
        pltpu.sync_copy(uid_hbm.at[pl.ds(base, BPW)], uidx_v)
        pltpu.sync_copy(iid_hbm.at[pl.ds(base, BPW)], iidx_v)
        cu = pltpu.async_copy(utab_hbm.at[uidx_v], urows_v, usem)
        ci = pltpu.async_copy(itab_hbm.at[iidx_v], irows_v, isem)
        pltpu.sync_copy(content_hbm.at[pl.ds(base, BPW), :], c_v)
        cu.wait()
        ci.wait()

        lanes = lax.iota(jnp.int32, L)

        def blk_body(blk, carry):
            acc = jnp.zeros((L,), jnp.float32)
            for j in range(L):
                b = blk * L + j
                p = (urows_v[b, :] + c_v[b, :]) * irows_v[b, :]
                score = jnp.sum(p, axis=0)
                acc = jnp.where(lanes == j, score, acc)
            s_v[pl.ds(blk * L, L)] = acc
            return carry

        lax.fori_loop(0, BPW // L, blk_body, 0)
        pltpu.sync_copy(s_v, out_hbm.at[pl.ds(base, BPW)])

    return _sc_score


def kernel(user_ids, item_ids, user_features, cf_user_table, cf_item_table,
           W1, b1, W2, b2):
    content = _mlp(user_features, W1, b1.reshape(1, H), W2, b2.reshape(1, E))
    return _make_sc_score()(user_ids, item_ids, content,
                            cf_user_table, cf_item_table)


# speedup vs baseline: 2.4722x; 2.4442x over previous
import functools

import jax
import jax.numpy as jnp
from jax import lax
from jax.experimental import pallas as pl
from jax.experimental.pallas import tpu as pltpu
from jax.experimental.pallas import tpu_sc as plsc

B = 16384
F = 128
H = 32
E = 16
NR = 1000000
NC, NS, L = 2, 16, 16
NW = NC * NS
MLP_BB = 2048

FULL_TILES = NR // 128
TAIL0 = FULL_TILES * 128
T = 8
UWT = -(-FULL_TILES // NS)
UCH = -(-UWT // T)
IWT = -(-FULL_TILES // NW)
ICH = -(-IWT // T)
LCAP = 1536
GCAP = LCAP // 8
SBK = 16
SBCAP = 256
DCAP = 128
SHARD = B // NS


def _mlp_body(uf_ref, w1_ref, b1_ref, w2_ref, b2_ref, out_ref):
    h = jnp.dot(uf_ref[...], w1_ref[...], preferred_element_type=jnp.float32)
    h = jnp.maximum(h + b1_ref[...], 0.0)
    out_ref[...] = (
        jnp.dot(h, w2_ref[...], preferred_element_type=jnp.float32) + b2_ref[...]
    )


_mlp = pl.pallas_call(
    _mlp_body,
    grid=(B // MLP_BB,),
    in_specs=[
        pl.BlockSpec((MLP_BB, F), lambda i: (i, 0)),
        pl.BlockSpec((F, H), lambda i: (0, 0)),
        pl.BlockSpec((1, H), lambda i: (0, 0)),
        pl.BlockSpec((H, E), lambda i: (0, 0)),
        pl.BlockSpec((1, E), lambda i: (0, 0)),
    ],
    out_specs=pl.BlockSpec((MLP_BB, E), lambda i: (i, 0)),
    out_shape=jax.ShapeDtypeStruct((B, E), jnp.float32),
)


@functools.cache
def _make_sc_score():
    mesh = plsc.VectorSubcoreMesh(
        core_axis_name="c", subcore_axis_name="s", num_cores=NC, num_subcores=NS
    )

    @functools.partial(
        pl.kernel,
        out_type=jax.ShapeDtypeStruct((NC * B,), jnp.float32),
        mesh=mesh,
        compiler_params=pltpu.CompilerParams(needs_layout_passes=False),
        scratch_types=[
            pltpu.VMEM((B,), jnp.int32),
            pltpu.VMEM((LCAP,), jnp.int32),
            pltpu.VMEM((LCAP,), jnp.int32),
            pltpu.VMEM((64,), jnp.int32),
            pltpu.VMEM((64,), jnp.int32),
            pltpu.VMEM((SBK * SBCAP,), jnp.int32),
            pltpu.VMEM((SBK * SBCAP,), jnp.int32),
            pltpu.VMEM((T, 8, 128), jnp.float32),
            pltpu.VMEM((T, 8, 128), jnp.float32),
            pltpu.VMEM((8, 128), jnp.float32),
            pltpu.VMEM((GCAP, 128), jnp.float32),
            pltpu.VMEM((NS * DCAP,), jnp.int32),
            pltpu.VMEM((2, 128), jnp.float32),
            pltpu.VMEM((L,), jnp.int32),
            pltpu.VMEM((DCAP // 8, 128), jnp.float32),
            pltpu.VMEM((DCAP,), jnp.int32),
            pltpu.VMEM((NC * NS * 128,), jnp.int32),
            pltpu.VMEM((128, 128), jnp.float32),
            pltpu.VMEM((128, 128), jnp.float32),
            pltpu.VMEM((SHARD,), jnp.float32),
            pltpu.HBM((NC * NS * NS * (DCAP // 8), 128), jnp.float32),
            pltpu.HBM((NC * NS * NS * DCAP,), jnp.int32),
            pltpu.HBM((NC * NS * 128,), jnp.int32),
            pltpu.SemaphoreType.DMA,
        ],
    )
    def _sc_score(uid_hbm, iid_hbm, content_hbm, utabT_hbm, itabT_hbm,
                  utail_hbm, itail_hbm, out_hbm,
                  idv, lid, lb, slid, slb, bkid, bkm,
                  band0, band1, tail_v, vals_v, mlist, rstage, bstage,
                  inval, inb, cnts_v, fu_v, sc_v, sout,
                  obx_vals, obx_b, obx_cnt, sem):
        cid = lax.axis_index("c")
        sid = lax.axis_index("s")
        wid = sid * NC + cid
        lanes = lax.iota(jnp.int32, L)

        def bfull(x):
            return jnp.full((L,), 1, jnp.int32) * x

        def scan_ids(t0, ntiles, strag_on):
            def body(k, carry):
                cnt, scnt = carry
                ids = idv[pl.ds(k * L, L)]
                t = jnp.right_shift(ids, 7)
                inr = (t >= t0) & (t < t0 + ntiles)
                bv = k * L + lanes
                plsc.store_compressed(lid.at[pl.ds(cnt, L)], ids, mask=inr)
                plsc.store_compressed(lb.at[pl.ds(cnt, L)], bv, mask=inr)
                cnt = cnt + plsc.all_reduce_population_count(inr)[0]
                st = (ids >= TAIL0) & strag_on
                plsc.store_compressed(slid.at[pl.ds(scnt, L)], ids, mask=st)
                plsc.store_compressed(slb.at[pl.ds(scnt, L)], bv, mask=st)
                scnt = scnt + plsc.all_reduce_population_count(st)[0]
                return (cnt, scnt)
            return lax.fori_loop(0, B // L, body,
                                 (jnp.int32(0), jnp.int32(0)))

        def extract_stragglers(cnt, scnt):
            def body(k, carry):
                idx = k * L + lanes
                m = idx < scnt
                zidx = jnp.where(m, idx, 0)
                ids = plsc.load_gather(slid, [zidx], mask=m)
                bv = plsc.load_gather(slb, [zidx], mask=m)
                r = jnp.where(m, ids - TAIL0, 0)
                mm = jnp.where(m, cnt + idx, 0)
                plsc.store_scatter(lid, [mm], ids, mask=m)
                plsc.store_scatter(lb, [mm], bv, mask=m)
                for e in range(E):
                    v = plsc.load_gather(
                        tail_v, [jnp.right_shift(r, 3), (r & 7) * E + e],
                        mask=m)
                    plsc.store_scatter(
                        vals_v,
                        [jnp.right_shift(mm, 3), (mm & 7) * E + e], v, mask=m)
                return carry
            lax.fori_loop(0, 4, body, 0)
            return cnt + scnt

        def super_bucket(t0, cnt, span):
            def body(k, carry):
                idx = k * L + lanes
                valid = idx < cnt
                zidx = jnp.where(valid, idx, 0)
                ids = plsc.load_gather(lid, [zidx], mask=valid)
                t = jnp.right_shift(ids, 7)
                c = jnp.right_shift(t - t0, 3)
                sb = jnp.clip(c // span, 0, SBK - 1)
                new = []
                for q in range(SBK):
                    qc = carry[q]
                    mq = valid & (sb == q)
                    plsc.store_compressed(
                        bkid.at[pl.ds(q * SBCAP + qc, L)], ids, mask=mq)
                    plsc.store_compressed(
                        bkm.at[pl.ds(q * SBCAP + qc, L)], idx, mask=mq)
                    new.append(qc + plsc.all_reduce_population_count(mq)[0])
                return tuple(new)
            return lax.fori_loop(0, LCAP // L, body,
                                 tuple(jnp.int32(0) for _ in range(SBK)))

        def sel(tup, q):
            out = tup[0]
            for i in range(1, len(tup)):
                out = jnp.where(q == i, tup[i], out)
            return out

        def stream_extract(tab_hbm, t0, ntiles, nch, span, bc):
            def chunk(c, carry):
                @pl.when(c * T < ntiles)
                def _():
                    def fire(j, carry2):
                        tg = jnp.minimum(t0 + c * T + j, FULL_TILES - 1)
                        pltpu.async_copy(
                            tab_hbm.at[pl.ds(0, 8), pl.ds(tg * 128, 128)],
                            band0.at[j], sem)
                        pltpu.async_copy(
                            tab_hbm.at[pl.ds(8, 8), pl.ds(tg * 128, 128)],
                            band1.at[j], sem)
                        return carry2
                    lax.fori_loop(0, T, fire, 0)
                    pltpu.make_async_copy(
                        tab_hbm.at[pl.ds(0, 8), pl.ds(0, T * 128)],
                        band0, sem).wait()
                    pltpu.make_async_copy(
                        tab_hbm.at[pl.ds(0, 8), pl.ds(0, T * 128)],
                        band1, sem).wait()
                    q = jnp.clip(c // span, 0, SBK - 1)
                    nq = sel(bc, q)

                    def grp(k, carry3):
                        idx = k * L + lanes
                        valid = idx < nq
                        zidx = jnp.where(valid, idx, 0)
                        ids = plsc.load_gather(bkid, [q * SBCAP + zidx],
                                               mask=valid)
                        t = jnp.right_shift(ids, 7)
                        mhit = valid & (t >= t0 + c * T) & (t < t0 + (c + 1) * T)
                        tl = jnp.where(mhit, t - (t0 + c * T), 0)
                        ll = jnp.where(mhit, ids & 127, 0)
                        mm = plsc.load_gather(bkm, [q * SBCAP + zidx],
                                              mask=mhit)
                        mm = jnp.where(mhit, mm, 0)
                        for e in range(E):
                            src = band0 if e < 8 else band1
                            v = plsc.load_gather(
                                src, [tl, bfull(e % 8), ll], mask=mhit)
                            plsc.store_scatter(
                                vals_v,
                                [jnp.right_shift(mm, 3), (mm & 7) * E + e],
                                v, mask=mhit)
                        return carry3
                    lax.fori_loop(0, jnp.right_shift(nq + L - 1, 4), grp, 0)
                return carry
            lax.fori_loop(0, nch, chunk, 0)

        def deliver(cnt):
            def mbody(k, carry):
                idx = k * L + lanes
                valid = idx < cnt
                zidx = jnp.where(valid, idx, 0)
                bv = plsc.load_gather(lb, [zidx], mask=valid)
                d = jnp.where(valid, jnp.right_shift(bv, 10), 0)
                new = []
                for j in range(NS):
                    jc = carry[j]
                    mj = valid & (d == j)
                    plsc.store_compressed(
                        mlist.at[pl.ds(j * DCAP + jc, L)], idx, mask=mj)
                    new.append(jc + plsc.all_reduce_population_count(mj)[0])
                return tuple(new)
            dc = lax.fori_loop(0, jnp.right_shift(cnt + L - 1, 4), mbody,
                               tuple(jnp.int32(0) for _ in range(NS)))

            cntrow = jnp.zeros((L,), jnp.int32)
            for j in range(NS):
                cntrow = jnp.where(lanes == j, dc[j], cntrow)
            bstage[pl.ds(0, L)] = cntrow
            pltpu.sync_copy(
                bstage, obx_cnt.at[pl.ds((cid * NS + sid) * 128, L)])

            def dst(j, carry):
                nj = sel(dc, j)

                def pk(k, carry2):
                    if True:
                        idx = k * L + lanes
                        valid = idx < nj
                        zidx = jnp.where(valid, idx, 0)
                        mm = plsc.load_gather(mlist, [j * DCAP + zidx],
                                              mask=valid)
                        mm = jnp.where(valid, mm, 0)
                        bv = plsc.load_gather(lb, [mm], mask=valid)
                        bstage[pl.ds(0, L)] = bv
                        pltpu.sync_copy(
                            bstage,
                            obx_b.at[pl.ds(
                                ((cid * NS + sid) * NS + j) * DCAP + k * L,
                                L)])
                        for e in range(E):
                            v = plsc.load_gather(
                                vals_v,
                                [jnp.right_shift(mm, 3), (mm & 7) * E + e],
                                mask=valid)
                            plsc.store_scatter(
                                rstage,
                                [jnp.right_shift(lanes, 3),
                                 (lanes & 7) * E + e], v, mask=valid)
                        pltpu.sync_copy(
                            rstage,
                            obx_vals.at[
                                pl.ds(((cid * NS + sid) * NS + j)
                                      * (DCAP // 8) + k * 2, 2), :])
                    return carry2
                lax.fori_loop(0, jnp.right_shift(nj + L - 1, 4), pk, 0)
                return carry
            lax.fori_loop(0, NS, dst, 0)

        def owner_counts():
            pltpu.sync_copy(obx_cnt, cnts_v)
            return plsc.load_gather(
                cnts_v, [(cid * NS + lanes) * 128 + sid])

        def owner_consume(cv, fold_user):
            def src_body(src, carry):
                n = plsc.load_gather(
                    cnts_v, [bfull((cid * NS + src) * 128) + sid])[0]
                pltpu.sync_copy(
                    obx_vals.at[
                        pl.ds(((cid * NS + src) * NS + sid) * (DCAP // 8),
                              DCAP // 8), :],
                    inval)
                pltpu.sync_copy(
                    obx_b.at[
                        pl.ds(((cid * NS + src) * NS + sid) * DCAP, DCAP)],
                    inb)

                def grp(k, carry2):
                    if True:
                        idx = k * L + lanes
                        valid = idx < n
                        zidx = jnp.where(valid, idx, 0)
                        bv = plsc.load_gather(inb, [zidx], mask=valid)
                        bl = jnp.where(valid, bv & (SHARD - 1), 0)
                        if fold_user:
                            for e in range(E):
                                v = plsc.load_gather(
                                    inval,
                                    [jnp.right_shift(zidx, 3),
                                     (zidx & 7) * E + e], mask=valid)
                                plsc.addupdate_scatter(
                                    fu_v,
                                    [jnp.right_shift(bl, 3),
                                     (bl & 7) * E + e], v, mask=valid)
                        else:
                            acc = jnp.zeros((L,), jnp.float32)
                            for e in range(E):
                                iv = plsc.load_gather(
                                    inval,
                                    [jnp.right_shift(zidx, 3),
                                     (zidx & 7) * E + e], mask=valid)
                                fv = plsc.load_gather(
                                    fu_v,
                                    [jnp.right_shift(bl, 3),
                                     (bl & 7) * E + e], mask=valid)
                                acc = acc + iv * fv
                            plsc.store_scatter(
                                sc_v,
                                [jnp.right_shift(bl, 3), (bl & 7) * E],
                                acc, mask=valid)
                    return carry2
                lax.fori_loop(0, jnp.right_shift(n + L - 1, 4), grp, 0)
                return carry
            lax.fori_loop(0, NS, src_body, 0)

        pltpu.sync_copy(uid_hbm, idv)
        pltpu.sync_copy(content_hbm.at[pl.ds(sid * 128, 128), :], fu_v)
        pltpu.sync_copy(utail_hbm, tail_v)
        ut0 = sid * UWT
        unt = jnp.minimum(UWT, FULL_TILES - ut0)
        span_u = (UCH + SBK - 1) // SBK
        cnt, scnt = scan_ids(ut0, unt, sid == NS - 1)
        bc = super_bucket(ut0, cnt, span_u)
        stream_extract(utabT_hbm, ut0, unt, UCH, span_u, bc)
        cnt = extract_stragglers(cnt, scnt)
        deliver(cnt)
        plsc.subcore_barrier()

        cv = owner_counts()
        owner_consume(cv, fold_user=True)
        plsc.subcore_barrier()

        pltpu.sync_copy(iid_hbm, idv)
        pltpu.sync_copy(itail_hbm, tail_v)

        def zero_sc(r, carry):
            z = jnp.zeros((L,), jnp.float32)
            for c8 in range(8):
                sc_v[r, pl.ds(c8 * L, L)] = z
            return carry
        lax.fori_loop(0, 128, zero_sc, 0)

        it0 = wid * IWT
        int_ = jnp.maximum(jnp.minimum(IWT, FULL_TILES - it0), 0)
        span_i = (ICH + SBK - 1) // SBK
        cnt, scnt = scan_ids(it0, int_, wid == NW - 1)
        bc = super_bucket(it0, cnt, span_i)
        stream_extract(itabT_hbm, it0, int_, ICH, span_i, bc)
        cnt = extract_stragglers(cnt, scnt)
        deliver(cnt)
        plsc.subcore_barrier()

        cv = owner_counts()
        owner_consume(cv, fold_user=False)

        def out_body(g, carry):
            b16 = g * L + lanes
            v = plsc.load_gather(
                sc_v, [jnp.right_shift(b16, 3), (b16 & 7) * E])
            sout[pl.ds(g * L, L)] = v
            return carry
        lax.fori_loop(0, SHARD // L, out_body, 0)
        pltpu.sync_copy(
            sout, out_hbm.at[pl.ds(cid * B + sid * SHARD, SHARD)])

    return _sc_score


def kernel(user_ids, item_ids, user_features, cf_user_table, cf_item_table,
           W1, b1, W2, b2):
    content = _mlp(user_features, W1, b1.reshape(1, H), W2, b2.reshape(1, E))
    out2 = _make_sc_score()(
        user_ids, item_ids, content.reshape(B * E // 128, 128),
        cf_user_table.T, cf_item_table.T,
        cf_user_table[TAIL0:].reshape(8, 128),
        cf_item_table[TAIL0:].reshape(8, 128))
    return out2[:B] + out2[B:]


# speedup vs baseline: 3.0073x; 1.2164x over previous
import functools

import jax
import jax.numpy as jnp
from jax import lax
from jax.experimental import pallas as pl
from jax.experimental.pallas import tpu as pltpu
from jax.experimental.pallas import tpu_sc as plsc

B = 16384
F = 128
H = 32
E = 16
NR = 1000000
NC, NS, L = 2, 16, 16
NW = NC * NS
MLP_BB = 2048

FULL_TILES = NR // 128
TAIL0 = FULL_TILES * 128
T = 8
UWT = -(-FULL_TILES // NS)
UCH = -(-UWT // T)
IWT = -(-FULL_TILES // NW)
ICH = -(-IWT // T)
LCAP = 1536
GCAP = LCAP // 8
SBK = 16
SBCAP = 256
DCAP = 128
SHARD = B // NS


def _mlp_body(uf_ref, w1_ref, b1_ref, w2_ref, b2_ref, out_ref):
    h = jnp.dot(uf_ref[...], w1_ref[...], preferred_element_type=jnp.float32)
    h = jnp.maximum(h + b1_ref[...], 0.0)
    out_ref[...] = (
        jnp.dot(h, w2_ref[...], preferred_element_type=jnp.float32) + b2_ref[...]
    )


_mlp = pl.pallas_call(
    _mlp_body,
    grid=(B // MLP_BB,),
    in_specs=[
        pl.BlockSpec((MLP_BB, F), lambda i: (i, 0)),
        pl.BlockSpec((F, H), lambda i: (0, 0)),
        pl.BlockSpec((1, H), lambda i: (0, 0)),
        pl.BlockSpec((H, E), lambda i: (0, 0)),
        pl.BlockSpec((1, E), lambda i: (0, 0)),
    ],
    out_specs=pl.BlockSpec((MLP_BB, E), lambda i: (i, 0)),
    out_shape=jax.ShapeDtypeStruct((B, E), jnp.float32),
)


@functools.cache
def _make_sc_score():
    mesh = plsc.VectorSubcoreMesh(
        core_axis_name="c", subcore_axis_name="s", num_cores=NC, num_subcores=NS
    )

    @functools.partial(
        pl.kernel,
        out_type=jax.ShapeDtypeStruct((NC * B,), jnp.float32),
        mesh=mesh,
        compiler_params=pltpu.CompilerParams(needs_layout_passes=False),
        scratch_types=[
            pltpu.VMEM((B,), jnp.int32),
            pltpu.VMEM((LCAP,), jnp.int32),
            pltpu.VMEM((LCAP,), jnp.int32),
            pltpu.VMEM((64,), jnp.int32),
            pltpu.VMEM((64,), jnp.int32),
            pltpu.VMEM((SBK * SBCAP,), jnp.int32),
            pltpu.VMEM((SBK * SBCAP,), jnp.int32),
            pltpu.VMEM((T, 8, 128), jnp.float32),
            pltpu.VMEM((T, 8, 128), jnp.float32),
            pltpu.VMEM((T, 8, 128), jnp.float32),
            pltpu.VMEM((T, 8, 128), jnp.float32),
            pltpu.VMEM((8, 128), jnp.float32),
            pltpu.VMEM((GCAP, 128), jnp.float32),
            pltpu.VMEM((NS * DCAP,), jnp.int32),
            pltpu.VMEM((2, 128), jnp.float32),
            pltpu.VMEM((L,), jnp.int32),
            pltpu.VMEM((DCAP // 8, 128), jnp.float32),
            pltpu.VMEM((DCAP,), jnp.int32),
            pltpu.VMEM((NC * NS * L,), jnp.int32),
            pltpu.VMEM((128, 128), jnp.float32),
            pltpu.VMEM((128, 128), jnp.float32),
            pltpu.VMEM((SHARD,), jnp.float32),
            pltpu.HBM((NC * NS * NS * (DCAP // 8), 128), jnp.float32),
            pltpu.HBM((NC * NS * NS * DCAP,), jnp.int32),
            pltpu.HBM((NC * NS * L,), jnp.int32),
            pltpu.SemaphoreType.DMA,
            pltpu.SemaphoreType.DMA,
        ],
    )
    def _sc_score(uid_hbm, iid_hbm, content_hbm, utabT_hbm, itabT_hbm,
                  utail_hbm, itail_hbm, out_hbm,
                  idv, lid, lb, slid, slb, bkid, bkm,
                  band0a, band1a, band0b, band1b,
                  tail_v, vals_v, mlist, rstage, bstage,
                  inval, inb, cnts_v, fu_v, sc_v, sout,
                  obx_vals, obx_b, obx_cnt, semA, semB):
        cid = lax.axis_index("c")
        sid = lax.axis_index("s")
        wid = sid * NC + cid
        lanes = lax.iota(jnp.int32, L)

        def bfull(x):
            return jnp.full((L,), 1, jnp.int32) * x

        def scan_ids(t0, ntiles, strag_on):
            def body(k, carry):
                cnt, scnt = carry
                ids = idv[pl.ds(k * L, L)]
                t = jnp.right_shift(ids, 7)
                inr = (t >= t0) & (t < t0 + ntiles)
                bv = k * L + lanes
                plsc.store_compressed(lid.at[pl.ds(cnt, L)], ids, mask=inr)
                plsc.store_compressed(lb.at[pl.ds(cnt, L)], bv, mask=inr)
                cnt = cnt + plsc.all_reduce_population_count(inr)[0]
                st = (ids >= TAIL0) & strag_on
                plsc.store_compressed(slid.at[pl.ds(scnt, L)], ids, mask=st)
                plsc.store_compressed(slb.at[pl.ds(scnt, L)], bv, mask=st)
                scnt = scnt + plsc.all_reduce_population_count(st)[0]
                return (cnt, scnt)
            return lax.fori_loop(0, B // L, body,
                                 (jnp.int32(0), jnp.int32(0)))

        def extract_stragglers(cnt, scnt):
            def body(k, carry):
                idx = k * L + lanes
                m = idx < scnt
                zidx = jnp.where(m, idx, 0)
                ids = plsc.load_gather(slid, [zidx], mask=m)
                bv = plsc.load_gather(slb, [zidx], mask=m)
                r = jnp.where(m, ids - TAIL0, 0)
                mm = jnp.where(m, cnt + idx, 0)
                plsc.store_scatter(lid, [mm], ids, mask=m)
                plsc.store_scatter(lb, [mm], bv, mask=m)
                for e in range(E):
                    v = plsc.load_gather(
                        tail_v, [jnp.right_shift(r, 3), (r & 7) * E + e],
                        mask=m)
                    plsc.store_scatter(
                        vals_v,
                        [jnp.right_shift(mm, 3), (mm & 7) * E + e], v, mask=m)
                return carry
            lax.fori_loop(0, 4, body, 0)
            return cnt + scnt

        def super_bucket(t0, cnt, span):
            def body(k, carry):
                idx = k * L + lanes
                valid = idx < cnt
                zidx = jnp.where(valid, idx, 0)
                ids = plsc.load_gather(lid, [zidx], mask=valid)
                t = jnp.right_shift(ids, 7)
                c = jnp.right_shift(t - t0, 3)
                sb = jnp.clip(c // span, 0, SBK - 1)
                new = []
                for q in range(SBK):
                    qc = carry[q]
                    mq = valid & (sb == q)
                    plsc.store_compressed(
                        bkid.at[pl.ds(q * SBCAP + qc, L)], ids, mask=mq)
                    plsc.store_compressed(
                        bkm.at[pl.ds(q * SBCAP + qc, L)], idx, mask=mq)
                    new.append(qc + plsc.all_reduce_population_count(mq)[0])
                return tuple(new)
            return lax.fori_loop(0, LCAP // L, body,
                                 tuple(jnp.int32(0) for _ in range(SBK)))

        def sel(tup, q):
            out = tup[0]
            for i in range(1, len(tup)):
                out = jnp.where(q == i, tup[i], out)
            return out

        def stream_extract(tab_hbm, t0, ntiles, nch, span, bc):
            def fire(c, b0, b1, sem):
                @pl.when(c * T < ntiles)
                def _():
                    def go(j, carry2):
                        tg = jnp.minimum(t0 + c * T + j, FULL_TILES - 1)
                        pltpu.async_copy(
                            tab_hbm.at[pl.ds(0, 8), pl.ds(tg * 128, 128)],
                            b0.at[j], sem)
                        pltpu.async_copy(
                            tab_hbm.at[pl.ds(8, 8), pl.ds(tg * 128, 128)],
                            b1.at[j], sem)
                        return carry2
                    lax.fori_loop(0, T, go, 0)

            def consume(c, b0, b1, sem):
                @pl.when(c * T < ntiles)
                def _():
                    pltpu.make_async_copy(
                        tab_hbm.at[pl.ds(0, 8), pl.ds(0, T * 128)],
                        b0, sem).wait()
                    pltpu.make_async_copy(
                        tab_hbm.at[pl.ds(0, 8), pl.ds(0, T * 128)],
                        b1, sem).wait()
                    q = jnp.clip(c // span, 0, SBK - 1)
                    nq = sel(bc, q)

                    def grp(k, carry3):
                        idx = k * L + lanes
                        valid = idx < nq
                        zidx = jnp.where(valid, idx, 0)
                        ids = plsc.load_gather(bkid, [q * SBCAP + zidx],
                                               mask=valid)
                        t = jnp.right_shift(ids, 7)
                        mhit = valid & (t >= t0 + c * T) & (t < t0 + (c + 1) * T)
                        tl = jnp.where(mhit, t - (t0 + c * T), 0)
                        ll = jnp.where(mhit, ids & 127, 0)
                        mm = plsc.load_gather(bkm, [q * SBCAP + zidx],
                                              mask=mhit)
                        mm = jnp.where(mhit, mm, 0)
                        for e in range(E):
                            src = b0 if e < 8 else b1
                            v = plsc.load_gather(
                                src, [tl, bfull(e % 8), ll], mask=mhit)
                            plsc.store_scatter(
                                vals_v,
                                [jnp.right_shift(mm, 3), (mm & 7) * E + e],
                                v, mask=mhit)
                        return carry3
                    lax.fori_loop(0, jnp.right_shift(nq + L - 1, 4), grp, 0)

            fire(0, band0a, band1a, semA)

            def pair(c2, carry):
                ca = 2 * c2
                cb = 2 * c2 + 1
                fire(cb, band0b, band1b, semB)
                consume(ca, band0a, band1a, semA)
                fire(cb + 1, band0a, band1a, semA)
                consume(cb, band0b, band1b, semB)
                return carry
            lax.fori_loop(0, (nch + 1) // 2, pair, 0)

        def deliver(cnt):
            def mbody(k, carry):
                idx = k * L + lanes
                valid = idx < cnt
                zidx = jnp.where(valid, idx, 0)
                bv = plsc.load_gather(lb, [zidx], mask=valid)
                d = jnp.where(valid, jnp.right_shift(bv, 10), 0)
                new = []
                for j in range(NS):
                    jc = carry[j]
                    mj = valid & (d == j)
                    plsc.store_compressed(
                        mlist.at[pl.ds(j * DCAP + jc, L)], idx, mask=mj)
                    new.append(jc + plsc.all_reduce_population_count(mj)[0])
                return tuple(new)
            dc = lax.fori_loop(0, jnp.right_shift(cnt + L - 1, 4), mbody,
                               tuple(jnp.int32(0) for _ in range(NS)))

            cntrow = jnp.zeros((L,), jnp.int32)
            for j in range(NS):
                cntrow = jnp.where(lanes == j, dc[j], cntrow)
            bstage[pl.ds(0, L)] = cntrow
            pltpu.sync_copy(
                bstage, obx_cnt.at[pl.ds((cid * NS + sid) * L, L)])

            def dst(j, carry):
                nj = sel(dc, j)

                def pk(k, carry2):
                    if True:
                        idx = k * L + lanes
                        valid = idx < nj
                        zidx = jnp.where(valid, idx, 0)
                        mm = plsc.load_gather(mlist, [j * DCAP + zidx],
                                              mask=valid)
                        mm = jnp.where(valid, mm, 0)
                        bv = plsc.load_gather(lb, [mm], mask=valid)
                        bstage[pl.ds(0, L)] = bv
                        pltpu.sync_copy(
                            bstage,
                            obx_b.at[pl.ds(
                                ((cid * NS + sid) * NS + j) * DCAP + k * L,
                                L)])
                        for e in range(E):
                            v = plsc.load_gather(
                                vals_v,
                                [jnp.right_shift(mm, 3), (mm & 7) * E + e],
                                mask=valid)
                            plsc.store_scatter(
                                rstage,
                                [jnp.right_shift(lanes, 3),
                                 (lanes & 7) * E + e], v, mask=valid)
                        pltpu.sync_copy(
                            rstage,
                            obx_vals.at[
                                pl.ds(((cid * NS + sid) * NS + j)
                                      * (DCAP // 8) + k * 2, 2), :])
                    return carry2
                lax.fori_loop(0, jnp.right_shift(nj + L - 1, 4), pk, 0)
                return carry
            lax.fori_loop(0, NS, dst, 0)

        def owner_counts():
            pltpu.sync_copy(obx_cnt, cnts_v)
            return plsc.load_gather(
                cnts_v, [(cid * NS + lanes) * L + sid])

        def owner_consume(cv, fold_user):
            def src_body(src, carry):
                n = plsc.load_gather(
                    cnts_v, [bfull((cid * NS + src) * L) + sid])[0]
                pltpu.sync_copy(
                    obx_vals.at[
                        pl.ds(((cid * NS + src) * NS + sid) * (DCAP // 8),
                              DCAP // 8), :],
                    inval)
                pltpu.sync_copy(
                    obx_b.at[
                        pl.ds(((cid * NS + src) * NS + sid) * DCAP, DCAP)],
                    inb)

                def grp(k, carry2):
                    if True:
                        idx = k * L + lanes
                        valid = idx < n
                        zidx = jnp.where(valid, idx, 0)
                        bv = plsc.load_gather(inb, [zidx], mask=valid)
                        bl = jnp.where(valid, bv & (SHARD - 1), 0)
                        if fold_user:
                            for e in range(E):
                                v = plsc.load_gather(
                                    inval,
                                    [jnp.right_shift(zidx, 3),
                                     (zidx & 7) * E + e], mask=valid)
                                plsc.addupdate_scatter(
                                    fu_v,
                                    [jnp.right_shift(bl, 3),
                                     (bl & 7) * E + e], v, mask=valid)
                        else:
                            acc = jnp.zeros((L,), jnp.float32)
                            for e in range(E):
                                iv = plsc.load_gather(
                                    inval,
                                    [jnp.right_shift(zidx, 3),
                                     (zidx & 7) * E + e], mask=valid)
                                fv = plsc.load_gather(
                                    fu_v,
                                    [jnp.right_shift(bl, 3),
                                     (bl & 7) * E + e], mask=valid)
                                acc = acc + iv * fv
                            plsc.store_scatter(
                                sc_v,
                                [jnp.right_shift(bl, 3), (bl & 7) * E],
                                acc, mask=valid)
                    return carry2
                lax.fori_loop(0, jnp.right_shift(n + L - 1, 4), grp, 0)
                return carry
            lax.fori_loop(0, NS, src_body, 0)

        pltpu.sync_copy(uid_hbm, idv)
        pltpu.sync_copy(content_hbm.at[pl.ds(sid * 128, 128), :], fu_v)
        pltpu.sync_copy(utail_hbm, tail_v)
        ut0 = sid * UWT
        unt = jnp.minimum(UWT, FULL_TILES - ut0)
        span_u = (UCH + SBK - 1) // SBK
        cnt, scnt = scan_ids(ut0, unt, sid == NS - 1)
        bc = super_bucket(ut0, cnt, span_u)
        stream_extract(utabT_hbm, ut0, unt, UCH, span_u, bc)
        cnt = extract_stragglers(cnt, scnt)
        deliver(cnt)
        plsc.subcore_barrier()

        cv = owner_counts()
        owner_consume(cv, fold_user=True)
        plsc.subcore_barrier()

        pltpu.sync_copy(iid_hbm, idv)
        pltpu.sync_copy(itail_hbm, tail_v)

        def zero_sc(r, carry):
            z = jnp.zeros((L,), jnp.float32)
            for c8 in range(8):
                sc_v[r, pl.ds(c8 * L, L)] = z
            return carry
        lax.fori_loop(0, 128, zero_sc, 0)

        it0 = wid * IWT
        int_ = jnp.maximum(jnp.minimum(IWT, FULL_TILES - it0), 0)
        span_i = (ICH + SBK - 1) // SBK
        cnt, scnt = scan_ids(it0, int_, wid == NW - 1)
        bc = super_bucket(it0, cnt, span_i)
        stream_extract(itabT_hbm, it0, int_, ICH, span_i, bc)
        cnt = extract_stragglers(cnt, scnt)
        deliver(cnt)
        plsc.subcore_barrier()

        cv = owner_counts()
        owner_consume(cv, fold_user=False)

        def out_body(g, carry):
            b16 = g * L + lanes
            v = plsc.load_gather(
                sc_v, [jnp.right_shift(b16, 3), (b16 & 7) * E])
            sout[pl.ds(g * L, L)] = v
            return carry
        lax.fori_loop(0, SHARD // L, out_body, 0)
        pltpu.sync_copy(
            sout, out_hbm.at[pl.ds(cid * B + sid * SHARD, SHARD)])

    return _sc_score


def kernel(user_ids, item_ids, user_features, cf_user_table, cf_item_table,
           W1, b1, W2, b2):
    content = _mlp(user_features, W1, b1.reshape(1, H), W2, b2.reshape(1, E))
    out2 = _make_sc_score()(
        user_ids, item_ids, content.reshape(B * E // 128, 128),
        cf_user_table.T, cf_item_table.T,
        cf_user_table[TAIL0:].reshape(8, 128),
        cf_item_table[TAIL0:].reshape(8, 128))
    return out2[:B] + out2[B:]
